# baseline (speedup 1.0000x reference)
import functools

import jax
import jax.numpy as jnp
from jax import lax
from jax.experimental import pallas as pl
from jax.experimental.pallas import tpu as pltpu
from jax.experimental.pallas import tpu_sc as plsc

N = 33300
NP = 33408
NP8 = NP // 8
D_IN = 128
F1 = 32
F2 = 16
NUM_NODES = 111
E = 532800
BLK = 128
NBLKT = 264
NT = 32
EP = 16 * NBLKT * BLK
EBLKS = EP // BLK
RPT = NP // 16
DEG_SPLIT = (152, 112)
AGG1_SPLIT = (168, 96)
AGG2_SPLIT = (144, 120)
G = 8
RING = 8
LOOK = 4

_mesh = plsc.VectorSubcoreMesh(core_axis_name="c", subcore_axis_name="s")
_sc_params = pltpu.CompilerParams(use_tc_tiling_on_sc=False)


def _zero16():
    return jnp.zeros((16,), jnp.float32)


def _fill_zeros(zb, width):
    def body(i, _):
        for k in range(width // 16):
            zb[i, pl.ds(k * 16, 16)] = _zero16()
        return 0
    lax.fori_loop(0, zb.shape[0], body, 0)


def _zero_acc_slice(zrow, acc, s, sem):
    _fill_zeros(zrow, zrow.shape[1])
    for j in range(16):
        pltpu.async_copy(zrow, acc.at[pl.ds(s * RPT + j * BLK, BLK)], sem)
    pltpu.async_copy(zrow.at[pl.ds(0, RPT - 16 * BLK)],
                     acc.at[pl.ds(s * RPT + 16 * BLK, RPT - 16 * BLK)], sem)
    for j in range(16):
        pltpu.make_async_copy(zrow, acc.at[pl.ds(s * RPT + j * BLK, BLK)],
                              sem).wait()
    pltpu.make_async_copy(zrow.at[pl.ds(0, RPT - 16 * BLK)],
                          acc.at[pl.ds(s * RPT + 16 * BLK, RPT - 16 * BLK)],
                          sem).wait()


def _deg_run(e2d, acc, ones, didx, ssem, blk0, nblk):
    pltpu.sync_copy(e2d.at[1, pl.ds(blk0, nblk)], didx.at[pl.ds(0, nblk)])

    def fire(b, _):
        pltpu.async_copy(ones, acc.at[didx.at[b]], ssem, add=True)
        return 0
    lax.fori_loop(0, nblk, fire, 0)

    def drain(b, _):
        pltpu.make_async_copy(ones, acc.at[didx.at[b]], ssem).wait()
        return 0
    lax.fori_loop(0, nblk, drain, 0)


def _deg_body(e2d, degp, acc, zrow, ones, didx, ssem):
    c = lax.axis_index("c")
    s = lax.axis_index("s")
    nb0, nb1 = DEG_SPLIT

    def fill_ones(i, _):
        ones[i, pl.ds(0, 16)] = _zero16() + 1.0
        return 0
    lax.fori_loop(0, BLK, fill_ones, 0)
    _zero_acc_slice(zrow, acc, s, ssem)
    plsc.subcore_barrier()

    @pl.when(c == 0)
    def _():
        _deg_run(e2d, acc, ones, didx, ssem, s * nb0, nb0)

    @pl.when(c == 1)
    def _():
        _deg_run(e2d, acc, ones, didx, ssem, 16 * nb0 + s * nb1, nb1)

    plsc.subcore_barrier()
    pltpu.sync_copy(acc.at[pl.ds(s * RPT, RPT)], degp.at[c, pl.ds(s * RPT, RPT)])


_deg_call = functools.partial(
    pl.kernel,
    mesh=_mesh,
    compiler_params=_sc_params,
    out_type=jax.ShapeDtypeStruct((2, NP, F2), jnp.float32),
    scratch_types=[
        pltpu.VMEM_SHARED((NP, F2), jnp.float32),
        pltpu.VMEM((BLK, F2), jnp.float32),
        pltpu.VMEM((BLK, F2), jnp.float32),
        pltpu.VMEM((DEG_SPLIT[0], BLK), jnp.int32),
        pltpu.SemaphoreType.DMA,
    ],
)(_deg_body)


def _agg_run(g, e2d, acc, sc, dc, rows, gsem, ssem, isem, blk0, nblk):
    ngroups = nblk // G

    def stage(grp, buf, sem_or_wait):
        src_sl = e2d.at[0, pl.ds(blk0 + grp * G, G)]
        dst_sl = e2d.at[1, pl.ds(blk0 + grp * G, G)]
        if sem_or_wait == "sync":
            pltpu.sync_copy(src_sl, sc[buf])
            pltpu.sync_copy(dst_sl, dc[buf])
        elif sem_or_wait == "fire":
            pltpu.async_copy(src_sl, sc[buf], isem)
            pltpu.async_copy(dst_sl, dc[buf], isem)
        else:
            pltpu.make_async_copy(src_sl, sc[buf], isem).wait()
            pltpu.make_async_copy(dst_sl, dc[buf], isem).wait()

    stage(0, 0, "sync")
    stage(1, 1, "fire")
    for k in range(LOOK):
        pltpu.async_copy(g.at[sc[0].at[k]], rows[k], gsem)

    def it(i, _):
        for gi in range(3):
            cur = gi
            nxt = (gi + 1) % 3
            old = (gi + 2) % 3
            for k in range(G):
                b = (i * 3 + gi) * G + k
                grp = (i * 3 + gi)

                @pl.when(b >= LOOK)
                def _():
                    cbuf = cur if k >= LOOK else old
                    pltpu.make_async_copy(
                        rows[(k + LOOK) % RING],
                        acc.at[dc[cbuf].at[(k + LOOK) % G]], ssem).wait()

                if k == LOOK:
                    @pl.when((grp + 2) * G < nblk)
                    def _():
                        stage(grp + 2, old, "fire")

                @pl.when(b + LOOK < nblk)
                def _():
                    if k + LOOK >= G:
                        if k == G - LOOK:
                            stage(grp + 1, nxt, "wait")
                        fbuf, frow = nxt, (k + LOOK) % G
                    else:
                        fbuf, frow = cur, k + LOOK
                    pltpu.async_copy(g.at[sc[fbuf].at[frow]],
                                     rows[(k + LOOK) % RING], gsem)

                pltpu.make_async_copy(g.at[sc[cur].at[k]], rows[k % RING],
                                      gsem).wait()
                pltpu.async_copy(rows[k % RING], acc.at[dc[cur].at[k]],
                                 ssem, add=True)
        return 0
    lax.fori_loop(0, ngroups // 3, it, 0)

    lbuf = (ngroups - 1) % 3
    for j in range(LOOK):
        b = nblk - LOOK + j
        pltpu.make_async_copy(rows[b % RING], acc.at[dc[lbuf].at[b % G]],
                              ssem).wait()


def _make_agg(F, split):
    nb0, nb1 = split

    def _agg_body(g, e2d, aggp, acc, s0, s1, s2, d0, d1, d2, *rest):
        rows, (gsem, ssem, isem) = list(rest[:RING]), rest[RING:]
        sc = [s0, s1, s2]
        dc = [d0, d1, d2]
        c = lax.axis_index("c")
        s = lax.axis_index("s")
        _zero_acc_slice(rows[0], acc, s, ssem)
        plsc.subcore_barrier()

        @pl.when(c == 0)
        def _():
            _agg_run(g, e2d, acc, sc, dc, rows, gsem, ssem, isem,
                     s * nb0, nb0)

        @pl.when(c == 1)
        def _():
            _agg_run(g, e2d, acc, sc, dc, rows, gsem, ssem, isem,
                     16 * nb0 + s * nb1, nb1)

        plsc.subcore_barrier()
        pltpu.sync_copy(acc.at[pl.ds(s * RPT, RPT)],
                        aggp.at[c, pl.ds(s * RPT, RPT)])

    return functools.partial(
        pl.kernel,
        mesh=_mesh,
        compiler_params=_sc_params,
        out_type=jax.ShapeDtypeStruct((2, NP, F), jnp.float32),
        scratch_types=[
            pltpu.VMEM_SHARED((NP, F), jnp.float32),
        ] + [pltpu.VMEM((G, BLK), jnp.int32)] * 6
          + [pltpu.VMEM((BLK, F), jnp.float32)] * RING + [
            pltpu.SemaphoreType.DMA,
            pltpu.SemaphoreType.DMA,
            pltpu.SemaphoreType.DMA,
        ],
    )(_agg_body)


_agg1_call = _make_agg(F1, AGG1_SPLIT)
_agg2_call = _make_agg(F2, AGG2_SPLIT)

RB8 = 2088


def _h1_body(x8r, w1br, h1r):
    h1r[...] = jnp.dot(x8r[...], w1br[...], preferred_element_type=jnp.float32)


def _h1(x8, W1b):
    return pl.pallas_call(
        _h1_body,
        grid=(2,),
        in_specs=[
            pl.BlockSpec((RB8, 8 * D_IN), lambda i: (i, 0)),
            pl.BlockSpec((8 * D_IN, 8 * F1), lambda i: (0, 0)),
        ],
        out_specs=pl.BlockSpec((RB8, 8 * F1), lambda i: (i, 0)),
        out_shape=jax.ShapeDtypeStruct((NP8, 8 * F1), jnp.float32),
    )(x8, W1b)


def _dis16(dpr):
    return lax.rsqrt(dpr[0] + dpr[1] + 1.0)


def _pre1b_body(h1r, dpr, q8r, g1r):
    rep32 = jnp.dot(_dis16(dpr), q8r[...], preferred_element_type=jnp.float32)
    g1r[...] = rep32 * h1r[...]


def _pre1b(h1p, degv, Q8):
    return pl.pallas_call(
        _pre1b_body,
        grid=(2,),
        in_specs=[
            pl.BlockSpec((RB8, 8 * F1), lambda i: (i, 0)),
            pl.BlockSpec((2, RB8, 8 * F2), lambda i: (0, i, 0)),
            pl.BlockSpec((8 * F2, 8 * F1), lambda i: (0, 0)),
        ],
        out_specs=pl.BlockSpec((RB8, 8 * F1), lambda i: (i, 0)),
        out_shape=jax.ShapeDtypeStruct((NP8, 8 * F1), jnp.float32),
    )(h1p, degv, Q8)


def _mid_body(apr, g1r, dpr, q8r, b1r, w2br, g2r):
    dis16 = _dis16(dpr)
    rep32 = jnp.dot(dis16, q8r[...], preferred_element_type=jnp.float32)
    out1 = jnp.maximum(rep32 * (apr[0] + apr[1] + g1r[...]) + b1r[...], 0.0)
    g2r[...] = dis16 * jnp.dot(out1, w2br[...],
                               preferred_element_type=jnp.float32)


def _mid(aggv1, g1p, degv, Q8, b1t, W2b):
    return pl.pallas_call(
        _mid_body,
        grid=(2,),
        in_specs=[
            pl.BlockSpec((2, RB8, 8 * F1), lambda i: (0, i, 0)),
            pl.BlockSpec((RB8, 8 * F1), lambda i: (i, 0)),
            pl.BlockSpec((2, RB8, 8 * F2), lambda i: (0, i, 0)),
            pl.BlockSpec((8 * F2, 8 * F1), lambda i: (0, 0)),
            pl.BlockSpec((1, 8 * F1), lambda i: (0, 0)),
            pl.BlockSpec((8 * F1, 8 * F2), lambda i: (0, 0)),
        ],
        out_specs=pl.BlockSpec((RB8, 8 * F2), lambda i: (i, 0)),
        out_shape=jax.ShapeDtypeStruct((NP8, 8 * F2), jnp.float32),
    )(aggv1, g1p, degv, Q8, b1t, W2b)


def _post2_body(apr, g2r, dpr, b2r, outr):
    dis16 = _dis16(dpr)
    outr[...] = jnp.maximum(
        dis16 * (apr[0] + apr[1] + g2r[...]) + b2r[...], 0.0)


def _post2(aggv2, g2p, degv, b2t):
    return pl.pallas_call(
        _post2_body,
        grid=(2,),
        in_specs=[
            pl.BlockSpec((2, RB8, 8 * F2), lambda i: (0, i, 0)),
            pl.BlockSpec((RB8, 8 * F2), lambda i: (i, 0)),
            pl.BlockSpec((2, RB8, 8 * F2), lambda i: (0, i, 0)),
            pl.BlockSpec((1, 8 * F2), lambda i: (0, 0)),
        ],
        out_specs=pl.BlockSpec((RB8, 8 * F2), lambda i: (i, 0)),
        out_shape=jax.ShapeDtypeStruct((NP8, 8 * F2), jnp.float32),
    )(aggv2, g2p, degv, b2t)


def _head_body(hr, wfcr, bfcr, wfc2r, bfc2r, outr):
    t = jnp.dot(hr[...], wfcr[...], preferred_element_type=jnp.float32)
    t = jnp.maximum(t + bfcr[...], 0.0)
    y = jnp.dot(t, wfc2r[...], preferred_element_type=jnp.float32)
    y = y + bfc2r[...]
    outr[...] = 1.0 / (1.0 + jnp.exp(-y))


def _head(hflat, Wfc, bfc, Wfc2, bfc2):
    return pl.pallas_call(
        _head_body,
        out_shape=jax.ShapeDtypeStruct((N // NUM_NODES, 1), jnp.float32),
    )(hflat, Wfc, bfc, Wfc2, bfc2)


def kernel(x, edge_index, W1, b1, W2, b2, Wfc, bfc, Wfc2, bfc2):
    f32 = jnp.float32
    ei = edge_index.astype(jnp.int32)
    e2d = jnp.pad(ei, ((0, 0), (0, EP - E)),
                  constant_values=N).reshape(2, EBLKS, BLK)
    x8 = jnp.pad(x, ((0, NP - N), (0, 0))).reshape(NP8, 8 * D_IN)

    eye16x2 = jnp.concatenate(
        [jnp.eye(16, dtype=f32), jnp.eye(16, dtype=f32)], axis=1)
    Q8 = jnp.kron(jnp.eye(8, dtype=f32), eye16x2)
    W1b = jnp.kron(jnp.eye(8, dtype=f32), W1)
    W2b = jnp.kron(jnp.eye(8, dtype=f32), W2)
    b1t = jnp.tile(b1, 8).reshape(1, 8 * F1)
    b2t = jnp.tile(b2, 8).reshape(1, 8 * F2)

    degp = _deg_call(e2d)
    degv = degp.reshape(2, NP8, 8 * F2)
    h1p = _h1(x8, W1b)
    g1p = _pre1b(h1p, degv, Q8)

    aggp1 = _agg1_call(g1p.reshape(NP, F1), e2d)
    g2p = _mid(aggp1.reshape(2, NP8, 8 * F1), g1p, degv, Q8, b1t, W2b)

    aggp2 = _agg2_call(g2p.reshape(NP, F2), e2d)
    out2p = _post2(aggp2.reshape(2, NP8, 8 * F2), g2p, degv, b2t)

    hflat = out2p.reshape(-1)[: N * F2].reshape(N // NUM_NODES,
                                                F2 * NUM_NODES)
    return _head(hflat, Wfc, bfc.reshape(1, 64), Wfc2, bfc2.reshape(1, 1))


# speedup vs baseline: 1.0315x; 1.0315x over previous
import functools

import jax
import jax.numpy as jnp
from jax import lax
from jax.experimental import pallas as pl
from jax.experimental.pallas import tpu as pltpu
from jax.experimental.pallas import tpu_sc as plsc

N = 33300
NP = 33408
NP8 = NP // 8
D_IN = 128
F1 = 32
F2 = 16
NUM_NODES = 111
E = 532800
BLK = 128
NBLKT = 264
NT = 32
EP = 16 * NBLKT * BLK
EBLKS = EP // BLK
RPT = NP // 16
DEG_SPLIT = (152, 112)
AGG1_SPLIT = (192, 72)
AGG2_SPLIT = (168, 96)
G = 8
RING = 8
LOOK0 = 4
LOOK1 = 3

_mesh = plsc.VectorSubcoreMesh(core_axis_name="c", subcore_axis_name="s")
_sc_params = pltpu.CompilerParams(use_tc_tiling_on_sc=False)


def _zero16():
    return jnp.zeros((16,), jnp.float32)


def _fill_zeros(zb, width):
    def body(i, _):
        for k in range(width // 16):
            zb[i, pl.ds(k * 16, 16)] = _zero16()
        return 0
    lax.fori_loop(0, zb.shape[0], body, 0)


def _zero_acc_slice(zrow, acc, s, sem):
    _fill_zeros(zrow, zrow.shape[1])
    for j in range(16):
        pltpu.async_copy(zrow, acc.at[pl.ds(s * RPT + j * BLK, BLK)], sem)
    pltpu.async_copy(zrow.at[pl.ds(0, RPT - 16 * BLK)],
                     acc.at[pl.ds(s * RPT + 16 * BLK, RPT - 16 * BLK)], sem)
    for j in range(16):
        pltpu.make_async_copy(zrow, acc.at[pl.ds(s * RPT + j * BLK, BLK)],
                              sem).wait()
    pltpu.make_async_copy(zrow.at[pl.ds(0, RPT - 16 * BLK)],
                          acc.at[pl.ds(s * RPT + 16 * BLK, RPT - 16 * BLK)],
                          sem).wait()


def _deg_run(e2d, acc, ones, didx, ssem, blk0, nblk):
    pltpu.sync_copy(e2d.at[1, pl.ds(blk0, nblk)], didx.at[pl.ds(0, nblk)])

    def fire(b, _):
        pltpu.async_copy(ones, acc.at[didx.at[b]], ssem, add=True)
        return 0
    lax.fori_loop(0, nblk, fire, 0)

    def drain(b, _):
        pltpu.make_async_copy(ones, acc.at[didx.at[b]], ssem).wait()
        return 0
    lax.fori_loop(0, nblk, drain, 0)


def _deg_body(e2d, degp, acc, zrow, ones, didx, ssem):
    c = lax.axis_index("c")
    s = lax.axis_index("s")
    nb0, nb1 = DEG_SPLIT

    def fill_ones(i, _):
        ones[i, pl.ds(0, 16)] = _zero16() + 1.0
        return 0
    lax.fori_loop(0, BLK, fill_ones, 0)
    _zero_acc_slice(zrow, acc, s, ssem)
    plsc.subcore_barrier()

    @pl.when(c == 0)
    def _():
        _deg_run(e2d, acc, ones, didx, ssem, s * nb0, nb0)

    @pl.when(c == 1)
    def _():
        _deg_run(e2d, acc, ones, didx, ssem, 16 * nb0 + s * nb1, nb1)

    plsc.subcore_barrier()
    pltpu.sync_copy(acc.at[pl.ds(s * RPT, RPT)], degp.at[c, pl.ds(s * RPT, RPT)])


_deg_call = functools.partial(
    pl.kernel,
    mesh=_mesh,
    compiler_params=_sc_params,
    out_type=jax.ShapeDtypeStruct((2, NP, F2), jnp.float32),
    scratch_types=[
        pltpu.VMEM_SHARED((NP, F2), jnp.float32),
        pltpu.VMEM((BLK, F2), jnp.float32),
        pltpu.VMEM((BLK, F2), jnp.float32),
        pltpu.VMEM((DEG_SPLIT[0], BLK), jnp.int32),
        pltpu.SemaphoreType.DMA,
    ],
)(_deg_body)


def _agg_run(g, e2d, acc, sc, dc, rows, gsem, ssem, isem, blk0, nblk, LOOK):
    ngroups = nblk // G

    def stage(grp, buf, sem_or_wait):
        src_sl = e2d.at[0, pl.ds(blk0 + grp * G, G)]
        dst_sl = e2d.at[1, pl.ds(blk0 + grp * G, G)]
        if sem_or_wait == "sync":
            pltpu.sync_copy(src_sl, sc[buf])
            pltpu.sync_copy(dst_sl, dc[buf])
        elif sem_or_wait == "fire":
            pltpu.async_copy(src_sl, sc[buf], isem)
            pltpu.async_copy(dst_sl, dc[buf], isem)
        else:
            pltpu.make_async_copy(src_sl, sc[buf], isem).wait()
            pltpu.make_async_copy(dst_sl, dc[buf], isem).wait()

    stage(0, 0, "sync")
    stage(1, 1, "fire")
    for k in range(LOOK):
        pltpu.async_copy(g.at[sc[0].at[k]], rows[k], gsem)

    def it(i, _):
        for gi in range(3):
            cur = gi
            nxt = (gi + 1) % 3
            old = (gi + 2) % 3
            for k in range(G):
                b = (i * 3 + gi) * G + k
                grp = (i * 3 + gi)

                @pl.when(b >= LOOK)
                def _():
                    cbuf = cur if k >= LOOK else old
                    pltpu.make_async_copy(
                        rows[(k - LOOK) % RING],
                        acc.at[dc[cbuf].at[(k - LOOK) % G]], ssem).wait()

                if k == LOOK:
                    @pl.when((grp + 2) * G < nblk)
                    def _():
                        stage(grp + 2, old, "fire")

                @pl.when(b + LOOK < nblk)
                def _():
                    if k + LOOK >= G:
                        if k == G - LOOK:
                            stage(grp + 1, nxt, "wait")
                        fbuf, frow = nxt, (k + LOOK) % G
                    else:
                        fbuf, frow = cur, k + LOOK
                    pltpu.async_copy(g.at[sc[fbuf].at[frow]],
                                     rows[(k + LOOK) % RING], gsem)

                pltpu.make_async_copy(g.at[sc[cur].at[k]], rows[k % RING],
                                      gsem).wait()
                pltpu.async_copy(rows[k % RING], acc.at[dc[cur].at[k]],
                                 ssem, add=True)
        return 0
    lax.fori_loop(0, ngroups // 3, it, 0)

    lbuf = (ngroups - 1) % 3
    for j in range(LOOK):
        b = nblk - LOOK + j
        pltpu.make_async_copy(rows[b % RING], acc.at[dc[lbuf].at[b % G]],
                              ssem).wait()


def _make_agg(F, split):
    nb0, nb1 = split

    def _agg_body(g, e2d, aggp, acc, s0, s1, s2, d0, d1, d2, *rest):
        rows, (gsem, ssem, isem) = list(rest[:RING]), rest[RING:]
        sc = [s0, s1, s2]
        dc = [d0, d1, d2]
        c = lax.axis_index("c")
        s = lax.axis_index("s")
        _zero_acc_slice(rows[0], acc, s, ssem)
        plsc.subcore_barrier()

        @pl.when(c == 0)
        def _():
            _agg_run(g, e2d, acc, sc, dc, rows, gsem, ssem, isem,
                     s * nb0, nb0, LOOK0)

        @pl.when(c == 1)
        def _():
            _agg_run(g, e2d, acc, sc, dc, rows, gsem, ssem, isem,
                     16 * nb0 + s * nb1, nb1, LOOK1)

        plsc.subcore_barrier()
        pltpu.sync_copy(acc.at[pl.ds(s * RPT, RPT)],
                        aggp.at[c, pl.ds(s * RPT, RPT)])

    return functools.partial(
        pl.kernel,
        mesh=_mesh,
        compiler_params=_sc_params,
        out_type=jax.ShapeDtypeStruct((2, NP, F), jnp.float32),
        scratch_types=[
            pltpu.VMEM_SHARED((NP, F), jnp.float32),
        ] + [pltpu.VMEM((G, BLK), jnp.int32)] * 6
          + [pltpu.VMEM((BLK, F), jnp.float32)] * RING + [
            pltpu.SemaphoreType.DMA,
            pltpu.SemaphoreType.DMA,
            pltpu.SemaphoreType.DMA,
        ],
    )(_agg_body)


_agg1_call = _make_agg(F1, AGG1_SPLIT)
_agg2_call = _make_agg(F2, AGG2_SPLIT)

RB8 = 2088


def _h1_body(x8r, w1br, h1r):
    h1r[...] = jnp.dot(x8r[...], w1br[...], preferred_element_type=jnp.float32)


def _h1(x8, W1b):
    return pl.pallas_call(
        _h1_body,
        grid=(2,),
        in_specs=[
            pl.BlockSpec((RB8, 8 * D_IN), lambda i: (i, 0)),
            pl.BlockSpec((8 * D_IN, 8 * F1), lambda i: (0, 0)),
        ],
        out_specs=pl.BlockSpec((RB8, 8 * F1), lambda i: (i, 0)),
        out_shape=jax.ShapeDtypeStruct((NP8, 8 * F1), jnp.float32),
    )(x8, W1b)


def _dis16(dpr):
    return lax.rsqrt(dpr[0] + dpr[1] + 1.0)


def _pre1b_body(h1r, dpr, q8r, g1r):
    rep32 = jnp.dot(_dis16(dpr), q8r[...], preferred_element_type=jnp.float32)
    g1r[...] = rep32 * h1r[...]


def _pre1b(h1p, degv, Q8):
    return pl.pallas_call(
        _pre1b_body,
        grid=(2,),
        in_specs=[
            pl.BlockSpec((RB8, 8 * F1), lambda i: (i, 0)),
            pl.BlockSpec((2, RB8, 8 * F2), lambda i: (0, i, 0)),
            pl.BlockSpec((8 * F2, 8 * F1), lambda i: (0, 0)),
        ],
        out_specs=pl.BlockSpec((RB8, 8 * F1), lambda i: (i, 0)),
        out_shape=jax.ShapeDtypeStruct((NP8, 8 * F1), jnp.float32),
    )(h1p, degv, Q8)


def _mid_body(apr, g1r, dpr, q8r, b1r, w2br, g2r):
    dis16 = _dis16(dpr)
    rep32 = jnp.dot(dis16, q8r[...], preferred_element_type=jnp.float32)
    out1 = jnp.maximum(rep32 * (apr[0] + apr[1] + g1r[...]) + b1r[...], 0.0)
    g2r[...] = dis16 * jnp.dot(out1, w2br[...],
                               preferred_element_type=jnp.float32)


def _mid(aggv1, g1p, degv, Q8, b1t, W2b):
    return pl.pallas_call(
        _mid_body,
        grid=(2,),
        in_specs=[
            pl.BlockSpec((2, RB8, 8 * F1), lambda i: (0, i, 0)),
            pl.BlockSpec((RB8, 8 * F1), lambda i: (i, 0)),
            pl.BlockSpec((2, RB8, 8 * F2), lambda i: (0, i, 0)),
            pl.BlockSpec((8 * F2, 8 * F1), lambda i: (0, 0)),
            pl.BlockSpec((1, 8 * F1), lambda i: (0, 0)),
            pl.BlockSpec((8 * F1, 8 * F2), lambda i: (0, 0)),
        ],
        out_specs=pl.BlockSpec((RB8, 8 * F2), lambda i: (i, 0)),
        out_shape=jax.ShapeDtypeStruct((NP8, 8 * F2), jnp.float32),
    )(aggv1, g1p, degv, Q8, b1t, W2b)


def _post2_body(apr, g2r, dpr, b2r, outr):
    dis16 = _dis16(dpr)
    outr[...] = jnp.maximum(
        dis16 * (apr[0] + apr[1] + g2r[...]) + b2r[...], 0.0)


def _post2(aggv2, g2p, degv, b2t):
    return pl.pallas_call(
        _post2_body,
        grid=(2,),
        in_specs=[
            pl.BlockSpec((2, RB8, 8 * F2), lambda i: (0, i, 0)),
            pl.BlockSpec((RB8, 8 * F2), lambda i: (i, 0)),
            pl.BlockSpec((2, RB8, 8 * F2), lambda i: (0, i, 0)),
            pl.BlockSpec((1, 8 * F2), lambda i: (0, 0)),
        ],
        out_specs=pl.BlockSpec((RB8, 8 * F2), lambda i: (i, 0)),
        out_shape=jax.ShapeDtypeStruct((NP8, 8 * F2), jnp.float32),
    )(aggv2, g2p, degv, b2t)


def _head_body(hr, wfcr, bfcr, wfc2r, bfc2r, outr):
    t = jnp.dot(hr[...], wfcr[...], preferred_element_type=jnp.float32)
    t = jnp.maximum(t + bfcr[...], 0.0)
    y = jnp.dot(t, wfc2r[...], preferred_element_type=jnp.float32)
    y = y + bfc2r[...]
    outr[...] = 1.0 / (1.0 + jnp.exp(-y))


def _head(hflat, Wfc, bfc, Wfc2, bfc2):
    return pl.pallas_call(
        _head_body,
        out_shape=jax.ShapeDtypeStruct((N // NUM_NODES, 1), jnp.float32),
    )(hflat, Wfc, bfc, Wfc2, bfc2)


def kernel(x, edge_index, W1, b1, W2, b2, Wfc, bfc, Wfc2, bfc2):
    f32 = jnp.float32
    ei = edge_index.astype(jnp.int32)
    e2d = jnp.pad(ei, ((0, 0), (0, EP - E)),
                  constant_values=N).reshape(2, EBLKS, BLK)
    x8 = jnp.pad(x, ((0, NP - N), (0, 0))).reshape(NP8, 8 * D_IN)

    eye16x2 = jnp.concatenate(
        [jnp.eye(16, dtype=f32), jnp.eye(16, dtype=f32)], axis=1)
    Q8 = jnp.kron(jnp.eye(8, dtype=f32), eye16x2)
    W1b = jnp.kron(jnp.eye(8, dtype=f32), W1)
    W2b = jnp.kron(jnp.eye(8, dtype=f32), W2)
    b1t = jnp.tile(b1, 8).reshape(1, 8 * F1)
    b2t = jnp.tile(b2, 8).reshape(1, 8 * F2)

    degp = _deg_call(e2d)
    degv = degp.reshape(2, NP8, 8 * F2)
    h1p = _h1(x8, W1b)
    g1p = _pre1b(h1p, degv, Q8)

    aggp1 = _agg1_call(g1p.reshape(NP, F1), e2d)
    g2p = _mid(aggp1.reshape(2, NP8, 8 * F1), g1p, degv, Q8, b1t, W2b)

    aggp2 = _agg2_call(g2p.reshape(NP, F2), e2d)
    out2p = _post2(aggp2.reshape(2, NP8, 8 * F2), g2p, degv, b2t)

    hflat = out2p.reshape(-1)[: N * F2].reshape(N // NUM_NODES,
                                                F2 * NUM_NODES)
    return _head(hflat, Wfc, bfc.reshape(1, 64), Wfc2, bfc2.reshape(1, 1))


# speedup vs baseline: 1.5842x; 1.5358x over previous
import functools

import jax
import jax.numpy as jnp
from jax import lax
from jax.experimental import pallas as pl
from jax.experimental.pallas import tpu as pltpu
from jax.experimental.pallas import tpu_sc as plsc

N = 33300
NP = 33408
NP8 = NP // 8
D_IN = 128
F1 = 32
F2 = 16
NUM_NODES = 111
E = 532800
BLK = 128
NBLKT = 264
NT = 32
EP = 16 * NBLKT * BLK
EBLKS = EP // BLK
RPT = NP // 16
DEG_SPLIT = (132, 132)
AGG1_SPLIT = (144, 120)
AGG2_SPLIT = (144, 120)
G = 8
RING = 8
LOOK0 = 4
LOOK1 = 4

_mesh = plsc.VectorSubcoreMesh(core_axis_name="c", subcore_axis_name="s")
_sc_params = pltpu.CompilerParams(use_tc_tiling_on_sc=False)


def _zero16():
    return jnp.zeros((16,), jnp.float32)


def _fill_zeros(zb, width):
    def body(i, _):
        for k in range(width // 16):
            zb[i, pl.ds(k * 16, 16)] = _zero16()
        return 0
    lax.fori_loop(0, zb.shape[0], body, 0)


def _zero_acc_slice(zrow, acc, s, sem):
    _fill_zeros(zrow, zrow.shape[1])
    for j in range(16):
        pltpu.async_copy(zrow, acc.at[pl.ds(s * RPT + j * BLK, BLK)], sem)
    pltpu.async_copy(zrow.at[pl.ds(0, RPT - 16 * BLK)],
                     acc.at[pl.ds(s * RPT + 16 * BLK, RPT - 16 * BLK)], sem)
    for j in range(16):
        pltpu.make_async_copy(zrow, acc.at[pl.ds(s * RPT + j * BLK, BLK)],
                              sem).wait()
    pltpu.make_async_copy(zrow.at[pl.ds(0, RPT - 16 * BLK)],
                          acc.at[pl.ds(s * RPT + 16 * BLK, RPT - 16 * BLK)],
                          sem).wait()


def _deg_run(e2d, acc, ones, didx, ssem, blk0, nblk):
    pltpu.sync_copy(e2d.at[1, pl.ds(blk0, nblk)], didx.at[pl.ds(0, nblk)])

    def fire(b, _):
        pltpu.async_copy(ones, acc.at[didx.at[b]], ssem, add=True)
        return 0
    lax.fori_loop(0, nblk, fire, 0)

    def drain(b, _):
        pltpu.make_async_copy(ones, acc.at[didx.at[b]], ssem).wait()
        return 0
    lax.fori_loop(0, nblk, drain, 0)


def _deg_body(e2d, degp, acc, zrow, ones, didx, ssem):
    c = lax.axis_index("c")
    s = lax.axis_index("s")
    nb0, nb1 = DEG_SPLIT

    def fill_ones(i, _):
        ones[i, pl.ds(0, 16)] = _zero16() + 1.0
        return 0
    lax.fori_loop(0, BLK, fill_ones, 0)
    _zero_acc_slice(zrow, acc, s, ssem)
    plsc.subcore_barrier()

    @pl.when(c == 0)
    def _():
        _deg_run(e2d, acc, ones, didx, ssem, s * nb0, nb0)

    @pl.when(c == 1)
    def _():
        _deg_run(e2d, acc, ones, didx, ssem, 16 * nb0 + s * nb1, nb1)

    plsc.subcore_barrier()
    pltpu.sync_copy(acc.at[pl.ds(s * RPT, RPT)], degp.at[c, pl.ds(s * RPT, RPT)])


_deg_call = functools.partial(
    pl.kernel,
    mesh=_mesh,
    compiler_params=_sc_params,
    out_type=jax.ShapeDtypeStruct((2, NP, F2), jnp.float32),
    scratch_types=[
        pltpu.VMEM_SHARED((NP, F2), jnp.float32),
        pltpu.VMEM((BLK, F2), jnp.float32),
        pltpu.VMEM((BLK, F2), jnp.float32),
        pltpu.VMEM((DEG_SPLIT[0], BLK), jnp.int32),
        pltpu.SemaphoreType.DMA,
    ],
)(_deg_body)


def _agg_run(g, e2d, acc, sc, dc, rows, gsem, ssem, isem, blk0, nblk, LOOK):
    ngroups = nblk // G

    def stage(grp, buf, sem_or_wait):
        src_sl = e2d.at[0, pl.ds(blk0 + grp * G, G)]
        dst_sl = e2d.at[1, pl.ds(blk0 + grp * G, G)]
        if sem_or_wait == "sync":
            pltpu.sync_copy(src_sl, sc[buf])
            pltpu.sync_copy(dst_sl, dc[buf])
        elif sem_or_wait == "fire":
            pltpu.async_copy(src_sl, sc[buf], isem)
            pltpu.async_copy(dst_sl, dc[buf], isem)
        else:
            pltpu.make_async_copy(src_sl, sc[buf], isem).wait()
            pltpu.make_async_copy(dst_sl, dc[buf], isem).wait()

    stage(0, 0, "sync")
    stage(1, 1, "fire")
    for k in range(LOOK):
        pltpu.async_copy(g.at[sc[0].at[k]], rows[k], gsem)

    def it(i, _):
        for gi in range(3):
            cur = gi
            nxt = (gi + 1) % 3
            old = (gi + 2) % 3
            for k in range(G):
                b = (i * 3 + gi) * G + k
                grp = (i * 3 + gi)

                @pl.when(b >= LOOK)
                def _():
                    cbuf = cur if k >= LOOK else old
                    pltpu.make_async_copy(
                        rows[(k - LOOK) % RING],
                        acc.at[dc[cbuf].at[(k - LOOK) % G]], ssem).wait()

                if k == LOOK:
                    @pl.when((grp + 2) * G < nblk)
                    def _():
                        stage(grp + 2, old, "fire")

                @pl.when(b + LOOK < nblk)
                def _():
                    if k + LOOK >= G:
                        if k == G - LOOK:
                            stage(grp + 1, nxt, "wait")
                        fbuf, frow = nxt, (k + LOOK) % G
                    else:
                        fbuf, frow = cur, k + LOOK
                    pltpu.async_copy(g.at[sc[fbuf].at[frow]],
                                     rows[(k + LOOK) % RING], gsem)

                pltpu.make_async_copy(g.at[sc[cur].at[k]], rows[k % RING],
                                      gsem).wait()
                pltpu.async_copy(rows[k % RING], acc.at[dc[cur].at[k]],
                                 ssem, add=True)
        return 0
    lax.fori_loop(0, ngroups // 3, it, 0)

    lbuf = (ngroups - 1) % 3
    for j in range(LOOK):
        b = nblk - LOOK + j
        pltpu.make_async_copy(rows[b % RING], acc.at[dc[lbuf].at[b % G]],
                              ssem).wait()


def _make_agg(F, split):
    nb0, nb1 = split

    def _agg_body(g, e2d, aggp, acc, s0, s1, s2, d0, d1, d2, *rest):
        rows, (gsem, ssem, isem) = list(rest[:RING]), rest[RING:]
        sc = [s0, s1, s2]
        dc = [d0, d1, d2]
        c = lax.axis_index("c")
        s = lax.axis_index("s")
        _zero_acc_slice(rows[0], acc, s, ssem)
        plsc.subcore_barrier()

        @pl.when(c == 0)
        def _():
            _agg_run(g, e2d, acc, sc, dc, rows, gsem, ssem, isem,
                     s * nb0, nb0, LOOK0)

        @pl.when(c == 1)
        def _():
            _agg_run(g, e2d, acc, sc, dc, rows, gsem, ssem, isem,
                     16 * nb0 + s * nb1, nb1, LOOK1)

        plsc.subcore_barrier()
        pltpu.sync_copy(acc.at[pl.ds(s * RPT, RPT)],
                        aggp.at[c, pl.ds(s * RPT, RPT)])

    return functools.partial(
        pl.kernel,
        mesh=_mesh,
        compiler_params=_sc_params,
        out_type=jax.ShapeDtypeStruct((2, NP, F), jnp.float32),
        scratch_types=[
            pltpu.VMEM_SHARED((NP, F), jnp.float32),
        ] + [pltpu.VMEM((G, BLK), jnp.int32)] * 6
          + [pltpu.VMEM((BLK, F), jnp.float32)] * RING + [
            pltpu.SemaphoreType.DMA,
            pltpu.SemaphoreType.DMA,
            pltpu.SemaphoreType.DMA,
        ],
    )(_agg_body)


_agg1_call = _make_agg(F1, AGG1_SPLIT)
_agg2_call = _make_agg(F2, AGG2_SPLIT)

RB8 = 2088


def _h1_body(x8r, w1br, h1r):
    h1r[...] = jnp.dot(x8r[...], w1br[...], preferred_element_type=jnp.float32)


def _h1(x8, W1b):
    return pl.pallas_call(
        _h1_body,
        grid=(2,),
        in_specs=[
            pl.BlockSpec((RB8, 8 * D_IN), lambda i: (i, 0)),
            pl.BlockSpec((8 * D_IN, 8 * F1), lambda i: (0, 0)),
        ],
        out_specs=pl.BlockSpec((RB8, 8 * F1), lambda i: (i, 0)),
        out_shape=jax.ShapeDtypeStruct((NP8, 8 * F1), jnp.float32),
    )(x8, W1b)


def _dis16(dpr):
    return lax.rsqrt(dpr[0] + dpr[1] + 1.0)


def _pre1b_body(h1r, dpr, q8r, g1r):
    rep32 = jnp.dot(_dis16(dpr), q8r[...], preferred_element_type=jnp.float32)
    g1r[...] = rep32 * h1r[...]


def _pre1b(h1p, degv, Q8):
    return pl.pallas_call(
        _pre1b_body,
        grid=(2,),
        in_specs=[
            pl.BlockSpec((RB8, 8 * F1), lambda i: (i, 0)),
            pl.BlockSpec((2, RB8, 8 * F2), lambda i: (0, i, 0)),
            pl.BlockSpec((8 * F2, 8 * F1), lambda i: (0, 0)),
        ],
        out_specs=pl.BlockSpec((RB8, 8 * F1), lambda i: (i, 0)),
        out_shape=jax.ShapeDtypeStruct((NP8, 8 * F1), jnp.float32),
    )(h1p, degv, Q8)


def _mid_body(apr, g1r, dpr, q8r, b1r, w2br, g2r):
    dis16 = _dis16(dpr)
    rep32 = jnp.dot(dis16, q8r[...], preferred_element_type=jnp.float32)
    out1 = jnp.maximum(rep32 * (apr[0] + apr[1] + g1r[...]) + b1r[...], 0.0)
    g2r[...] = dis16 * jnp.dot(out1, w2br[...],
                               preferred_element_type=jnp.float32)


def _mid(aggv1, g1p, degv, Q8, b1t, W2b):
    return pl.pallas_call(
        _mid_body,
        grid=(2,),
        in_specs=[
            pl.BlockSpec((2, RB8, 8 * F1), lambda i: (0, i, 0)),
            pl.BlockSpec((RB8, 8 * F1), lambda i: (i, 0)),
            pl.BlockSpec((2, RB8, 8 * F2), lambda i: (0, i, 0)),
            pl.BlockSpec((8 * F2, 8 * F1), lambda i: (0, 0)),
            pl.BlockSpec((1, 8 * F1), lambda i: (0, 0)),
            pl.BlockSpec((8 * F1, 8 * F2), lambda i: (0, 0)),
        ],
        out_specs=pl.BlockSpec((RB8, 8 * F2), lambda i: (i, 0)),
        out_shape=jax.ShapeDtypeStruct((NP8, 8 * F2), jnp.float32),
    )(aggv1, g1p, degv, Q8, b1t, W2b)


def _post2_body(apr, g2r, dpr, b2r, outr):
    dis16 = _dis16(dpr)
    outr[...] = jnp.maximum(
        dis16 * (apr[0] + apr[1] + g2r[...]) + b2r[...], 0.0)


def _post2(aggv2, g2p, degv, b2t):
    return pl.pallas_call(
        _post2_body,
        grid=(2,),
        in_specs=[
            pl.BlockSpec((2, RB8, 8 * F2), lambda i: (0, i, 0)),
            pl.BlockSpec((RB8, 8 * F2), lambda i: (i, 0)),
            pl.BlockSpec((2, RB8, 8 * F2), lambda i: (0, i, 0)),
            pl.BlockSpec((1, 8 * F2), lambda i: (0, 0)),
        ],
        out_specs=pl.BlockSpec((RB8, 8 * F2), lambda i: (i, 0)),
        out_shape=jax.ShapeDtypeStruct((NP8, 8 * F2), jnp.float32),
    )(aggv2, g2p, degv, b2t)


def _head_body(hr, wfcr, bfcr, wfc2r, bfc2r, outr):
    t = jnp.dot(hr[...], wfcr[...], preferred_element_type=jnp.float32)
    t = jnp.maximum(t + bfcr[...], 0.0)
    y = jnp.dot(t, wfc2r[...], preferred_element_type=jnp.float32)
    y = y + bfc2r[...]
    outr[...] = 1.0 / (1.0 + jnp.exp(-y))


def _head(hflat, Wfc, bfc, Wfc2, bfc2):
    return pl.pallas_call(
        _head_body,
        out_shape=jax.ShapeDtypeStruct((N // NUM_NODES, 1), jnp.float32),
    )(hflat, Wfc, bfc, Wfc2, bfc2)


def kernel(x, edge_index, W1, b1, W2, b2, Wfc, bfc, Wfc2, bfc2):
    f32 = jnp.float32
    ei = edge_index.astype(jnp.int32)
    pad_idx = N + jnp.arange(EP - E, dtype=jnp.int32) % (NP - N)
    e2d = jnp.concatenate(
        [ei, jnp.stack([pad_idx, pad_idx])], axis=1).reshape(2, EBLKS, BLK)
    x8 = jnp.pad(x, ((0, NP - N), (0, 0))).reshape(NP8, 8 * D_IN)

    eye16x2 = jnp.concatenate(
        [jnp.eye(16, dtype=f32), jnp.eye(16, dtype=f32)], axis=1)
    Q8 = jnp.kron(jnp.eye(8, dtype=f32), eye16x2)
    W1b = jnp.kron(jnp.eye(8, dtype=f32), W1)
    W2b = jnp.kron(jnp.eye(8, dtype=f32), W2)
    b1t = jnp.tile(b1, 8).reshape(1, 8 * F1)
    b2t = jnp.tile(b2, 8).reshape(1, 8 * F2)

    degp = _deg_call(e2d)
    degv = degp.reshape(2, NP8, 8 * F2)
    h1p = _h1(x8, W1b)
    g1p = _pre1b(h1p, degv, Q8)

    aggp1 = _agg1_call(g1p.reshape(NP, F1), e2d)
    g2p = _mid(aggp1.reshape(2, NP8, 8 * F1), g1p, degv, Q8, b1t, W2b)

    aggp2 = _agg2_call(g2p.reshape(NP, F2), e2d)
    out2p = _post2(aggp2.reshape(2, NP8, 8 * F2), g2p, degv, b2t)

    hflat = out2p.reshape(-1)[: N * F2].reshape(N // NUM_NODES,
                                                F2 * NUM_NODES)
    return _head(hflat, Wfc, bfc.reshape(1, 64), Wfc2, bfc2.reshape(1, 1))
